# baseline (speedup 1.0000x reference)
import functools

import jax
import jax.numpy as jnp
from jax import lax
from jax.experimental import pallas as pl
from jax.experimental.pallas import tpu as pltpu
from jax.experimental.pallas import tpu_sc as plsc

N = 10000
NU = 4000
H = 128
E = 320000
B = 4096
NC = 2
NS = 16
NPAD = 10240
ROWS_PER_TILE = NPAD // NS
EDGES_PER_TILE = E // NS
K = 128
NCHUNK = 160
CB = 40
NBLOCK = NCHUNK // CB
ACCR = N + NS
DEGW = 16
BLK = 1000
NBLK = N // BLK
UBLK = NU // BLK
PER = B // (NC * NS)


W1 = 624


def _sc_agg_body(with_deg, xs_tab, xt_tab, es_s, es_t, ed_s, ed_t, *rest):
    if with_deg:
        (agg_out, deg_out, acc, dacc, sidx, didx, rows0, rows1, zbufd,
         onesb, sem0, sem1) = rest
    else:
        agg_out, acc, sidx, didx, rows0, rows1, sem0, sem1 = rest
    c = lax.axis_index("c")
    s = lax.axis_index("s")

    z16 = jnp.zeros((16,), jnp.float32)
    z32 = jnp.zeros((32,), jnp.bfloat16)

    def zrow(r, _):
        for j in range(H // 32):
            rows0[r, pl.ds(j * 32, 32)] = z32
        return 0

    lax.fori_loop(0, K, zrow, 0)
    rb = s * W1
    for t in range(W1 // K):
        pltpu.sync_copy(rows0, acc.at[pl.ds(rb + t * K, K)])
    pltpu.sync_copy(rows0.at[pl.ds(0, W1 % K)],
                    acc.at[pl.ds(rb + (W1 // K) * K, W1 % K)])

    @pl.when(s == NS - 1)
    def _():
        pltpu.sync_copy(rows0.at[pl.ds(0, N - NS * W1)],
                        acc.at[pl.ds(NS * W1, N - NS * W1)])

    if with_deg:
        o16 = jnp.ones((16,), jnp.float32)

        def zdrow(r, _):
            zbufd[r, pl.ds(0, DEGW)] = z16
            onesb[r, pl.ds(0, DEGW)] = o16
            return 0

        lax.fori_loop(0, K, zdrow, 0)
        for t in range(W1 // K):
            pltpu.sync_copy(zbufd, dacc.at[pl.ds(rb + t * K, K)])
        pltpu.sync_copy(zbufd.at[pl.ds(0, W1 % K)],
                        dacc.at[pl.ds(rb + (W1 // K) * K, W1 % K)])

        @pl.when(s == NS - 1)
        def _():
            pltpu.sync_copy(zbufd.at[pl.ds(0, N - NS * W1)],
                            dacc.at[pl.ds(NS * W1, N - NS * W1)])

    plsc.subcore_barrier()

    def run(tab, es, ed):
        def block(bi, _):
            pltpu.sync_copy(es.at[s, bi], sidx)
            pltpu.sync_copy(ed.at[s, bi], didx)
            pltpu.async_copy(tab.at[sidx.at[0]], rows0, sem0)

            def pair(j, _):
                i0 = 2 * j
                pltpu.async_copy(tab.at[sidx.at[i0 + 1]], rows1, sem1)
                pltpu.make_async_copy(tab.at[pl.ds(0, K)], rows0,
                                      sem0).wait()
                pltpu.sync_copy(rows0, acc.at[didx.at[i0]], add=True)
                if with_deg:
                    pltpu.sync_copy(onesb, dacc.at[didx.at[i0]], add=True)

                @pl.when(i0 + 2 < CB)
                def _():
                    pltpu.async_copy(tab.at[sidx.at[i0 + 2]], rows0, sem0)

                pltpu.make_async_copy(tab.at[pl.ds(0, K)], rows1,
                                      sem1).wait()
                pltpu.sync_copy(rows1, acc.at[didx.at[i0 + 1]], add=True)
                if with_deg:
                    pltpu.sync_copy(onesb, dacc.at[didx.at[i0 + 1]],
                                    add=True)
                return 0

            lax.fori_loop(0, CB // 2, pair, 0)
            return 0

        lax.fori_loop(0, NBLOCK, block, 0)

    @pl.when(c == 0)
    def _():
        run(xs_tab, es_s, ed_s)

    @pl.when(c == 1)
    def _():
        run(xt_tab, es_t, ed_t)

    plsc.subcore_barrier()

    pltpu.sync_copy(acc.at[pl.ds(rb, W1)], agg_out.at[pl.ds(c * N + rb, W1)])

    @pl.when(s == NS - 1)
    def _():
        pltpu.sync_copy(acc.at[pl.ds(NS * W1, N - NS * W1)],
                        agg_out.at[pl.ds(c * N + NS * W1, N - NS * W1)])

    if with_deg:
        pltpu.sync_copy(dacc.at[pl.ds(rb, W1)],
                        deg_out.at[pl.ds(c * N + rb, W1)])

        @pl.when(s == NS - 1)
        def _():
            pltpu.sync_copy(dacc.at[pl.ds(NS * W1, N - NS * W1)],
                            deg_out.at[pl.ds(c * N + NS * W1, N - NS * W1)])


def _make_sc_agg(with_deg):
    mesh = plsc.VectorSubcoreMesh(core_axis_name="c", subcore_axis_name="s")
    out_type = [jax.ShapeDtypeStruct((NC * N, H), jnp.bfloat16)]
    scratch = [
        pltpu.VMEM_SHARED((ACCR, H), jnp.bfloat16),
        pltpu.VMEM((CB, K), jnp.int32),
        pltpu.VMEM((CB, K), jnp.int32),
        pltpu.VMEM((K, H), jnp.bfloat16),
        pltpu.VMEM((K, H), jnp.bfloat16),
        pltpu.SemaphoreType.DMA,
        pltpu.SemaphoreType.DMA,
    ]
    if with_deg:
        out_type.append(jax.ShapeDtypeStruct((NC * N, DEGW), jnp.float32))
        scratch = [
            pltpu.VMEM_SHARED((ACCR, H), jnp.bfloat16),
            pltpu.VMEM_SHARED((ACCR, DEGW), jnp.float32),
            pltpu.VMEM((CB, K), jnp.int32),
            pltpu.VMEM((CB, K), jnp.int32),
            pltpu.VMEM((K, H), jnp.bfloat16),
            pltpu.VMEM((K, H), jnp.bfloat16),
            pltpu.VMEM((K, DEGW), jnp.float32),
            pltpu.VMEM((K, DEGW), jnp.float32),
            pltpu.SemaphoreType.DMA,
            pltpu.SemaphoreType.DMA,
        ]
    return pl.kernel(
        functools.partial(_sc_agg_body, with_deg),
        out_type=tuple(out_type) if with_deg else out_type[0],
        mesh=mesh,
        scratch_types=scratch,
        compiler_params=pltpu.CompilerParams(use_tc_tiling_on_sc=False),
    )


def _sc_pred_body(u_hbm, v_hbm, l0, l1, out_hbm, u_v, v_v, l0v, l1v, outv):
    c = lax.axis_index("c")
    s = lax.axis_index("s")
    wid = c * NS + s
    pltpu.sync_copy(u_hbm, u_v)
    pltpu.sync_copy(v_hbm, v_v)
    base = wid * PER
    pltpu.sync_copy(l0.at[pl.ds(base, PER)], l0v)
    pltpu.sync_copy(l1.at[pl.ds(base, PER)], l1v)
    for j in range(PER // 16):
        i0 = l0v[pl.ds(j * 16, 16)]
        i1 = l1v[pl.ds(j * 16, 16)]
        uu = plsc.load_gather(u_v, [i0])
        vv = plsc.load_gather(v_v, [i1])
        sc = uu + vv
        sc = jnp.maximum(sc, 0.01 * sc)
        outv[pl.ds(j * 16, 16)] = 1.0 / (1.0 + jnp.exp(-sc))
    pltpu.sync_copy(outv, out_hbm.at[pl.ds(base, PER)])


def _make_sc_pred():
    mesh = plsc.VectorSubcoreMesh(core_axis_name="c", subcore_axis_name="s")
    return pl.kernel(
        _sc_pred_body,
        out_type=jax.ShapeDtypeStruct((B,), jnp.float32),
        mesh=mesh,
        scratch_types=[
            pltpu.VMEM((N,), jnp.float32),
            pltpu.VMEM((N,), jnp.float32),
            pltpu.VMEM((PER,), jnp.int32),
            pltpu.VMEM((PER,), jnp.int32),
            pltpu.VMEM((PER,), jnp.float32),
        ],
        compiler_params=pltpu.CompilerParams(use_tc_tiling_on_sc=False,
                                             needs_layout_passes=False),
    )


def _dot(a, b):
    return jnp.dot(a, b, preferred_element_type=jnp.float32)


def _tc_layer1_body(xs_r, xt_r, as_r, at_r, ds_r, dt_r, wss, wns, bs_r,
                    wst, wnt, bt_r, mws, mwt, mb_r, os_r, ot_r, osb_r,
                    otb_r):
    i = pl.program_id(0)
    recs = 1.0 / jnp.maximum(ds_r[...][:, 0:1], 1.0)
    rect = 1.0 / jnp.maximum(dt_r[...][:, 0:1], 1.0)
    ys = jnp.maximum(
        _dot(xs_r[...], wss[...])
        + _dot(as_r[...].astype(jnp.float32) * recs, wns[...])
        + bs_r[...], 0.0)
    yt = jnp.maximum(
        _dot(xt_r[...], wst[...])
        + _dot(at_r[...].astype(jnp.float32) * rect, wnt[...])
        + bt_r[...], 0.0)

    @pl.when(i < UBLK)
    def _():
        u = _dot(ys, mws[...]) + _dot(yt, mwt[...]) + mb_r[...]
        os_r[...] = u
        ot_r[...] = u
        ub = u.astype(jnp.bfloat16)
        osb_r[...] = ub
        otb_r[...] = ub

    @pl.when(i >= UBLK)
    def _():
        os_r[...] = ys
        ot_r[...] = yt
        osb_r[...] = ys.astype(jnp.bfloat16)
        otb_r[...] = yt.astype(jnp.bfloat16)


def _tc_layer2_body(emb_r, xs_r, xt_r, as_r, at_r, ds_r, dt_r, wss, wns,
                    bs_r, wst, wnt, bt_r, mws, mwt, mb_r, a0, a1, a2, bv,
                    uv_r):
    i = pl.program_id(0)
    recs = 1.0 / jnp.maximum(ds_r[...][:, 0:1], 1.0)
    rect = 1.0 / jnp.maximum(dt_r[...][:, 0:1], 1.0)
    ys = jnp.maximum(
        _dot(xs_r[...], wss[...])
        + _dot(as_r[...].astype(jnp.float32) * recs, wns[...])
        + bs_r[...], 0.0)
    yt = jnp.maximum(
        _dot(xt_r[...], wst[...])
        + _dot(at_r[...].astype(jnp.float32) * rect, wnt[...])
        + bt_r[...], 0.0)
    part = _dot(emb_r[...], a0[...]) + _dot(xs_r[...], a1[...]) + bv[...]

    @pl.when(i < UBLK)
    def _():
        u = _dot(ys, mws[...]) + _dot(yt, mwt[...]) + mb_r[...]
        uv_r[...] = part + _dot(u, a2[...])

    @pl.when(i >= UBLK)
    def _():
        uv_r[...] = part + _dot(ys, a2[...])


def _row_spec(off=0):
    return pl.BlockSpec((BLK, H), lambda i: (i + off, 0))


def _deg_spec(off=0):
    return pl.BlockSpec((BLK, DEGW), lambda i: (i + off, 0))


def _w_spec(r, c):
    return pl.BlockSpec((r, c), lambda i: (0, 0))


def _tc_layer1(xs, xt, aggs, aggt, degs, degt, wss, wns, bs, wst, wnt, bt,
               mws, mwt, mb):
    in_specs = [_row_spec(), _row_spec(), _row_spec(), _row_spec(NBLK),
                _deg_spec(), _deg_spec(NBLK)] + [
        _w_spec(H, H), _w_spec(H, H), _w_spec(1, H),
        _w_spec(H, H), _w_spec(H, H), _w_spec(1, H),
        _w_spec(H, H), _w_spec(H, H), _w_spec(1, H),
    ]
    out_specs = [_row_spec(), _row_spec(), _row_spec(), _row_spec()]
    return pl.pallas_call(
        _tc_layer1_body,
        grid=(NBLK,),
        in_specs=in_specs,
        out_specs=out_specs,
        out_shape=[jax.ShapeDtypeStruct((N, H), jnp.float32)] * 2
        + [jax.ShapeDtypeStruct((N, H), jnp.bfloat16)] * 2,
    )(xs, xt, aggs, aggt, degs, degt, wss, wns, bs, wst, wnt, bt, mws, mwt,
      mb)


def _tc_layer2(emb, xs, xt, aggs, aggt, degs, degt, wss, wns, bs, wst, wnt,
               bt, mws, mwt, mb, a0, a1, a2, bv):
    in_specs = [_row_spec(), _row_spec(), _row_spec(), _row_spec(),
                _row_spec(NBLK), _deg_spec(), _deg_spec(NBLK)] + [
        _w_spec(H, H), _w_spec(H, H), _w_spec(1, H),
        _w_spec(H, H), _w_spec(H, H), _w_spec(1, H),
        _w_spec(H, H), _w_spec(H, H), _w_spec(1, H),
        _w_spec(H, 8), _w_spec(H, 8), _w_spec(H, 8), _w_spec(1, 8),
    ]
    return pl.pallas_call(
        _tc_layer2_body,
        grid=(NBLK,),
        in_specs=in_specs,
        out_specs=pl.BlockSpec((BLK, 8), lambda i: (i, 0)),
        out_shape=jax.ShapeDtypeStruct((N, 8), jnp.float32),
    )(emb, xs, xt, aggs, aggt, degs, degt, wss, wns, bs, wst, wnt, bt, mws,
      mwt, mb, a0, a1, a2, bv)


def kernel(embedding, src_Wself, src_Wneigh, src_b, tgt_Wself, tgt_Wneigh,
           tgt_b, mix_W, mix_b, pred_W, pred_b, source_edge_index,
           target_edge_index, link):
    f32 = jnp.float32
    emb = embedding.astype(f32)
    se = source_edge_index.astype(jnp.int32)
    te = target_edge_index.astype(jnp.int32)
    lk = link.astype(jnp.int32)

    npad = NCHUNK * K - EDGES_PER_TILE

    trash = N + jnp.arange(NS, dtype=jnp.int32)[:, None]

    def _tile_pad(a, fill):
        pad = jnp.broadcast_to(fill, (NS, npad)).astype(jnp.int32)
        return jnp.concatenate(
            [a.reshape(NS, EDGES_PER_TILE), pad], axis=1
        ).reshape(NS, NBLOCK, CB, K)

    es_s = _tile_pad(se[0], 0)
    es_t = _tile_pad(te[0], 0)
    ed_s = _tile_pad(se[1], trash)
    ed_t = _tile_pad(te[1], trash)

    mws = mix_W[:, :H, :]
    mwt = mix_W[:, H:, :]
    pW = pred_W.astype(f32)
    a_list = []
    for i in range(3):
        a = jnp.concatenate(
            [pW[H * i:H * (i + 1)], pW[384 + H * i:384 + H * (i + 1)]],
            axis=1)
        a_list.append(jnp.pad(a, ((0, 0), (0, 6))))
    bv = jnp.zeros((1, 8), f32).at[0, 0].set(pred_b[0].astype(f32))

    emb_bf = emb.astype(jnp.bfloat16)
    agg1, deg = _make_sc_agg(True)(emb_bf, emb_bf, es_s, es_t, ed_s, ed_t)
    xs, xt, xs_bf, xt_bf = _tc_layer1(
        emb, emb, agg1, agg1, deg, deg,
        src_Wself[0], src_Wneigh[0], src_b[0].reshape(1, H),
        tgt_Wself[0], tgt_Wneigh[0], tgt_b[0].reshape(1, H),
        mws[0], mwt[0], mix_b[0].reshape(1, H))

    agg2 = _make_sc_agg(False)(xs_bf, xt_bf, es_s, es_t, ed_s, ed_t)
    uv = _tc_layer2(
        emb, xs, xt, agg2, agg2, deg, deg,
        src_Wself[1], src_Wneigh[1], src_b[1].reshape(1, H),
        tgt_Wself[1], tgt_Wneigh[1], tgt_b[1].reshape(1, H),
        mws[1], mwt[1], mix_b[1].reshape(1, H),
        a_list[0], a_list[1], a_list[2], bv)

    u = uv[:, 0]
    v = uv[:, 1]
    out = _make_sc_pred()(u, v, lk[0], lk[1])
    return out.reshape(B, 1)


# speedup vs baseline: 2.3805x; 2.3805x over previous
import functools

import jax
import jax.numpy as jnp
from jax import lax
from jax.experimental import pallas as pl
from jax.experimental.pallas import tpu as pltpu
from jax.experimental.pallas import tpu_sc as plsc

N = 10000
NU = 4000
H = 128
E = 320000
B = 4096
NC = 2
NS = 16
NPAD = 10240
ROWS_PER_TILE = NPAD // NS
EDGES_PER_TILE = E // NS
K = 80
NCHUNK = EDGES_PER_TILE // K
CB = 50
NBLOCK = NCHUNK // CB
ACCR = N
DEGW = 16
BLK = 1000
NBLK = N // BLK
UBLK = NU // BLK
PER = B // (NC * NS)


W1 = 624


def _sc_agg_body(with_deg, xs_tab, xt_tab, es_s, es_t, ed_s, ed_t, *rest):
    if with_deg:
        (agg_out, deg_out, acc, dacc, sidx, didx, rows0, rows1, rows2,
         rows3, zbufd, onesb, sem0, sem1, sem2, sem3) = rest
    else:
        (agg_out, acc, sidx, didx, rows0, rows1, rows2, rows3, sem0, sem1,
         sem2, sem3) = rest
    rows_bufs = (rows0, rows1, rows2, rows3)
    gsems = (sem0, sem1, sem2, sem3)
    c = lax.axis_index("c")
    s = lax.axis_index("s")

    z16 = jnp.zeros((16,), jnp.float32)
    z32 = jnp.zeros((32,), jnp.bfloat16)

    def zrow(r, _):
        for j in range(H // 32):
            rows0[r, pl.ds(j * 32, 32)] = z32
        return 0

    lax.fori_loop(0, K, zrow, 0)
    rb = s * W1
    for t in range(W1 // K):
        pltpu.sync_copy(rows0, acc.at[pl.ds(rb + t * K, K)])
    pltpu.sync_copy(rows0.at[pl.ds(0, W1 % K)],
                    acc.at[pl.ds(rb + (W1 // K) * K, W1 % K)])

    @pl.when(s == NS - 1)
    def _():
        pltpu.sync_copy(rows0.at[pl.ds(0, N - NS * W1)],
                        acc.at[pl.ds(NS * W1, N - NS * W1)])

    if with_deg:
        o16 = jnp.ones((16,), jnp.float32)

        def zdrow(r, _):
            zbufd[r, pl.ds(0, DEGW)] = z16
            onesb[r, pl.ds(0, DEGW)] = o16
            return 0

        lax.fori_loop(0, K, zdrow, 0)
        for t in range(W1 // K):
            pltpu.sync_copy(zbufd, dacc.at[pl.ds(rb + t * K, K)])
        pltpu.sync_copy(zbufd.at[pl.ds(0, W1 % K)],
                        dacc.at[pl.ds(rb + (W1 // K) * K, W1 % K)])

        @pl.when(s == NS - 1)
        def _():
            pltpu.sync_copy(zbufd.at[pl.ds(0, N - NS * W1)],
                            dacc.at[pl.ds(NS * W1, N - NS * W1)])

    plsc.subcore_barrier()

    def run(tab, es, ed):
        pltpu.sync_copy(es.at[s], sidx)
        pltpu.sync_copy(ed.at[s], didx)
        for b in range(4):
            pltpu.async_copy(tab.at[sidx.at[b]], rows_bufs[b], gsems[b])

        def quad(j, _):
            i0 = 4 * j
            for b in range(4):
                i = i0 + b
                pltpu.make_async_copy(tab.at[pl.ds(0, K)], rows_bufs[b],
                                      gsems[b]).wait()
                pltpu.sync_copy(rows_bufs[b], acc.at[didx.at[i]], add=True)
                if with_deg:
                    pltpu.sync_copy(onesb, dacc.at[didx.at[i]], add=True)

                @pl.when(i + 4 < NCHUNK)
                def _():
                    pltpu.async_copy(tab.at[sidx.at[i + 4]], rows_bufs[b],
                                     gsems[b])
            return 0

        lax.fori_loop(0, NCHUNK // 4, quad, 0)
        for b in range(NCHUNK % 4):
            i = (NCHUNK // 4) * 4 + b
            pltpu.make_async_copy(tab.at[pl.ds(0, K)], rows_bufs[b],
                                  gsems[b]).wait()
            pltpu.sync_copy(rows_bufs[b], acc.at[didx.at[i]], add=True)
            if with_deg:
                pltpu.sync_copy(onesb, dacc.at[didx.at[i]], add=True)

    @pl.when(c == 0)
    def _():
        run(xs_tab, es_s, ed_s)

    @pl.when(c == 1)
    def _():
        run(xt_tab, es_t, ed_t)

    plsc.subcore_barrier()

    pltpu.sync_copy(acc.at[pl.ds(rb, W1)], agg_out.at[pl.ds(c * N + rb, W1)])

    @pl.when(s == NS - 1)
    def _():
        pltpu.sync_copy(acc.at[pl.ds(NS * W1, N - NS * W1)],
                        agg_out.at[pl.ds(c * N + NS * W1, N - NS * W1)])

    if with_deg:
        pltpu.sync_copy(dacc.at[pl.ds(rb, W1)],
                        deg_out.at[pl.ds(c * N + rb, W1)])

        @pl.when(s == NS - 1)
        def _():
            pltpu.sync_copy(dacc.at[pl.ds(NS * W1, N - NS * W1)],
                            deg_out.at[pl.ds(c * N + NS * W1, N - NS * W1)])


def _make_sc_agg(with_deg):
    mesh = plsc.VectorSubcoreMesh(core_axis_name="c", subcore_axis_name="s")
    out_type = [jax.ShapeDtypeStruct((NC * N, H), jnp.bfloat16)]
    scratch = [
        pltpu.VMEM_SHARED((ACCR, H), jnp.bfloat16),
        pltpu.VMEM((NCHUNK, K), jnp.int32),
        pltpu.VMEM((NCHUNK, K), jnp.int32),
        pltpu.VMEM((K, H), jnp.bfloat16),
        pltpu.VMEM((K, H), jnp.bfloat16),
        pltpu.VMEM((K, H), jnp.bfloat16),
        pltpu.VMEM((K, H), jnp.bfloat16),
        pltpu.SemaphoreType.DMA,
        pltpu.SemaphoreType.DMA,
        pltpu.SemaphoreType.DMA,
        pltpu.SemaphoreType.DMA,
    ]
    if with_deg:
        out_type.append(jax.ShapeDtypeStruct((NC * N, DEGW), jnp.float32))
        scratch = [
            pltpu.VMEM_SHARED((ACCR, H), jnp.bfloat16),
            pltpu.VMEM_SHARED((ACCR, DEGW), jnp.float32),
            pltpu.VMEM((NCHUNK, K), jnp.int32),
            pltpu.VMEM((NCHUNK, K), jnp.int32),
            pltpu.VMEM((K, H), jnp.bfloat16),
            pltpu.VMEM((K, H), jnp.bfloat16),
            pltpu.VMEM((K, H), jnp.bfloat16),
            pltpu.VMEM((K, H), jnp.bfloat16),
            pltpu.VMEM((K, DEGW), jnp.float32),
            pltpu.VMEM((K, DEGW), jnp.float32),
            pltpu.SemaphoreType.DMA,
            pltpu.SemaphoreType.DMA,
            pltpu.SemaphoreType.DMA,
            pltpu.SemaphoreType.DMA,
        ]
    return pl.kernel(
        functools.partial(_sc_agg_body, with_deg),
        out_type=tuple(out_type) if with_deg else out_type[0],
        mesh=mesh,
        scratch_types=scratch,
        compiler_params=pltpu.CompilerParams(use_tc_tiling_on_sc=False),
    )


def _sc_pred_body(u_hbm, v_hbm, l0, l1, out_hbm, u_v, v_v, l0v, l1v, outv):
    c = lax.axis_index("c")
    s = lax.axis_index("s")
    wid = c * NS + s
    pltpu.sync_copy(u_hbm, u_v)
    pltpu.sync_copy(v_hbm, v_v)
    base = wid * PER
    pltpu.sync_copy(l0.at[pl.ds(base, PER)], l0v)
    pltpu.sync_copy(l1.at[pl.ds(base, PER)], l1v)
    for j in range(PER // 16):
        i0 = l0v[pl.ds(j * 16, 16)]
        i1 = l1v[pl.ds(j * 16, 16)]
        uu = plsc.load_gather(u_v, [i0])
        vv = plsc.load_gather(v_v, [i1])
        sc = uu + vv
        sc = jnp.maximum(sc, 0.01 * sc)
        outv[pl.ds(j * 16, 16)] = 1.0 / (1.0 + jnp.exp(-sc))
    pltpu.sync_copy(outv, out_hbm.at[pl.ds(base, PER)])


def _make_sc_pred():
    mesh = plsc.VectorSubcoreMesh(core_axis_name="c", subcore_axis_name="s")
    return pl.kernel(
        _sc_pred_body,
        out_type=jax.ShapeDtypeStruct((B,), jnp.float32),
        mesh=mesh,
        scratch_types=[
            pltpu.VMEM((N,), jnp.float32),
            pltpu.VMEM((N,), jnp.float32),
            pltpu.VMEM((PER,), jnp.int32),
            pltpu.VMEM((PER,), jnp.int32),
            pltpu.VMEM((PER,), jnp.float32),
        ],
        compiler_params=pltpu.CompilerParams(use_tc_tiling_on_sc=False,
                                             needs_layout_passes=False),
    )


def _dot(a, b):
    return jnp.dot(a, b, preferred_element_type=jnp.float32)


def _tc_layer1_body(xs_r, xt_r, as_r, at_r, ds_r, dt_r, wss, wns, bs_r,
                    wst, wnt, bt_r, mws, mwt, mb_r, os_r, ot_r, osb_r,
                    otb_r):
    i = pl.program_id(0)
    recs = 1.0 / jnp.maximum(ds_r[...][:, 0:1], 1.0)
    rect = 1.0 / jnp.maximum(dt_r[...][:, 0:1], 1.0)
    ys = jnp.maximum(
        _dot(xs_r[...], wss[...])
        + _dot(as_r[...].astype(jnp.float32) * recs, wns[...])
        + bs_r[...], 0.0)
    yt = jnp.maximum(
        _dot(xt_r[...], wst[...])
        + _dot(at_r[...].astype(jnp.float32) * rect, wnt[...])
        + bt_r[...], 0.0)

    @pl.when(i < UBLK)
    def _():
        u = _dot(ys, mws[...]) + _dot(yt, mwt[...]) + mb_r[...]
        os_r[...] = u
        ot_r[...] = u
        ub = u.astype(jnp.bfloat16)
        osb_r[...] = ub
        otb_r[...] = ub

    @pl.when(i >= UBLK)
    def _():
        os_r[...] = ys
        ot_r[...] = yt
        osb_r[...] = ys.astype(jnp.bfloat16)
        otb_r[...] = yt.astype(jnp.bfloat16)


def _tc_layer2_body(emb_r, xs_r, xt_r, as_r, at_r, ds_r, dt_r, wss, wns,
                    bs_r, wst, wnt, bt_r, mws, mwt, mb_r, a0, a1, a2, bv,
                    uv_r):
    i = pl.program_id(0)
    recs = 1.0 / jnp.maximum(ds_r[...][:, 0:1], 1.0)
    rect = 1.0 / jnp.maximum(dt_r[...][:, 0:1], 1.0)
    ys = jnp.maximum(
        _dot(xs_r[...], wss[...])
        + _dot(as_r[...].astype(jnp.float32) * recs, wns[...])
        + bs_r[...], 0.0)
    yt = jnp.maximum(
        _dot(xt_r[...], wst[...])
        + _dot(at_r[...].astype(jnp.float32) * rect, wnt[...])
        + bt_r[...], 0.0)
    part = _dot(emb_r[...], a0[...]) + _dot(xs_r[...], a1[...]) + bv[...]

    @pl.when(i < UBLK)
    def _():
        u = _dot(ys, mws[...]) + _dot(yt, mwt[...]) + mb_r[...]
        uv_r[...] = part + _dot(u, a2[...])

    @pl.when(i >= UBLK)
    def _():
        uv_r[...] = part + _dot(ys, a2[...])


def _row_spec(off=0):
    return pl.BlockSpec((BLK, H), lambda i: (i + off, 0))


def _deg_spec(off=0):
    return pl.BlockSpec((BLK, DEGW), lambda i: (i + off, 0))


def _w_spec(r, c):
    return pl.BlockSpec((r, c), lambda i: (0, 0))


def _tc_layer1(xs, xt, aggs, aggt, degs, degt, wss, wns, bs, wst, wnt, bt,
               mws, mwt, mb):
    in_specs = [_row_spec(), _row_spec(), _row_spec(), _row_spec(NBLK),
                _deg_spec(), _deg_spec(NBLK)] + [
        _w_spec(H, H), _w_spec(H, H), _w_spec(1, H),
        _w_spec(H, H), _w_spec(H, H), _w_spec(1, H),
        _w_spec(H, H), _w_spec(H, H), _w_spec(1, H),
    ]
    out_specs = [_row_spec(), _row_spec(), _row_spec(), _row_spec()]
    return pl.pallas_call(
        _tc_layer1_body,
        grid=(NBLK,),
        in_specs=in_specs,
        out_specs=out_specs,
        out_shape=[jax.ShapeDtypeStruct((N, H), jnp.float32)] * 2
        + [jax.ShapeDtypeStruct((N, H), jnp.bfloat16)] * 2,
    )(xs, xt, aggs, aggt, degs, degt, wss, wns, bs, wst, wnt, bt, mws, mwt,
      mb)


def _tc_layer2(emb, xs, xt, aggs, aggt, degs, degt, wss, wns, bs, wst, wnt,
               bt, mws, mwt, mb, a0, a1, a2, bv):
    in_specs = [_row_spec(), _row_spec(), _row_spec(), _row_spec(),
                _row_spec(NBLK), _deg_spec(), _deg_spec(NBLK)] + [
        _w_spec(H, H), _w_spec(H, H), _w_spec(1, H),
        _w_spec(H, H), _w_spec(H, H), _w_spec(1, H),
        _w_spec(H, H), _w_spec(H, H), _w_spec(1, H),
        _w_spec(H, 8), _w_spec(H, 8), _w_spec(H, 8), _w_spec(1, 8),
    ]
    return pl.pallas_call(
        _tc_layer2_body,
        grid=(NBLK,),
        in_specs=in_specs,
        out_specs=pl.BlockSpec((BLK, 8), lambda i: (i, 0)),
        out_shape=jax.ShapeDtypeStruct((N, 8), jnp.float32),
    )(emb, xs, xt, aggs, aggt, degs, degt, wss, wns, bs, wst, wnt, bt, mws,
      mwt, mb, a0, a1, a2, bv)


def kernel(embedding, src_Wself, src_Wneigh, src_b, tgt_Wself, tgt_Wneigh,
           tgt_b, mix_W, mix_b, pred_W, pred_b, source_edge_index,
           target_edge_index, link):
    f32 = jnp.float32
    emb = embedding.astype(f32)
    se = source_edge_index.astype(jnp.int32)
    te = target_edge_index.astype(jnp.int32)
    lk = link.astype(jnp.int32)

    esh = (NS, NCHUNK, K)
    es_s = se[0].reshape(esh)
    es_t = te[0].reshape(esh)
    ed_s = se[1].reshape(esh)
    ed_t = te[1].reshape(esh)

    mws = mix_W[:, :H, :]
    mwt = mix_W[:, H:, :]
    pW = pred_W.astype(f32)
    a_list = []
    for i in range(3):
        a = jnp.concatenate(
            [pW[H * i:H * (i + 1)], pW[384 + H * i:384 + H * (i + 1)]],
            axis=1)
        a_list.append(jnp.pad(a, ((0, 0), (0, 6))))
    bv = jnp.zeros((1, 8), f32).at[0, 0].set(pred_b[0].astype(f32))

    emb_bf = emb.astype(jnp.bfloat16)
    agg1, deg = _make_sc_agg(True)(emb_bf, emb_bf, es_s, es_t, ed_s, ed_t)
    xs, xt, xs_bf, xt_bf = _tc_layer1(
        emb, emb, agg1, agg1, deg, deg,
        src_Wself[0], src_Wneigh[0], src_b[0].reshape(1, H),
        tgt_Wself[0], tgt_Wneigh[0], tgt_b[0].reshape(1, H),
        mws[0], mwt[0], mix_b[0].reshape(1, H))

    agg2 = _make_sc_agg(False)(xs_bf, xt_bf, es_s, es_t, ed_s, ed_t)
    uv = _tc_layer2(
        emb, xs, xt, agg2, agg2, deg, deg,
        src_Wself[1], src_Wneigh[1], src_b[1].reshape(1, H),
        tgt_Wself[1], tgt_Wneigh[1], tgt_b[1].reshape(1, H),
        mws[1], mwt[1], mix_b[1].reshape(1, H),
        a_list[0], a_list[1], a_list[2], bv)

    u = uv[:, 0]
    v = uv[:, 1]
    out = _make_sc_pred()(u, v, lk[0], lk[1])
    return out.reshape(B, 1)
